# Initial kernel scaffold; baseline (speedup 1.0000x reference)
import jax
import jax.numpy as jnp
from jax.experimental import pallas as pl


def kernel(x, edge_index, Ws, bs, gammas, betas):
    raise NotImplementedError("write your pallas kernel here")


# speedup vs baseline: 18.7544x; 18.7544x over previous
import dataclasses
import functools

import jax
import jax.numpy as jnp
from jax import lax
from jax.experimental import pallas as pl
from jax.experimental.pallas import tpu as pltpu
from jax.experimental.pallas import tpu_sc as plsc

N = 10000
D = 128
E = 320000
NUM_LAYERS = 3
LN_EPS = 1e-5

NC = 2
NS = 16
CHUNK = 128
N_PAD = 10240
E_PAD = 327680
N_CHUNKS = E_PAD // CHUNK
CPT = N_CHUNKS // (NC * NS)
STRIP = N_PAD // NS
HROWS = N_PAD // CHUNK

_sc_mesh = plsc.VectorSubcoreMesh(core_axis_name="c", subcore_axis_name="s")

_cp = pltpu.CompilerParams()
if "needs_layout_passes" in pltpu.CompilerParams.__dataclass_fields__:
    _cp = dataclasses.replace(_cp, needs_layout_passes=False)

_Z16 = functools.partial(jnp.zeros, (16,), jnp.float32)


def _deg_body(dst_hbm, out_hbm, dstbuf, hist, idc, zbuf, acc_sh):
    c = lax.axis_index("c")
    s = lax.axis_index("s")
    hpt = 8

    @pl.loop(0, HROWS)
    def _zhist(i):
        @pl.loop(0, CHUNK // 16)
        def _zh2(j):
            hist[i, pl.ds(j * 16, 16)] = _Z16()

    @pl.loop(0, hpt)
    def _zrow(i):
        @pl.loop(0, CHUNK // 16)
        def _zr2(j):
            zbuf[i, pl.ds(j * 16, 16)] = _Z16()

    @pl.loop(0, HROWS // 16)
    def _iot(k):
        idc[pl.ds(k * 16, 16)] = lax.iota(jnp.int32, 16) + k * 16

    @pl.when(s < HROWS // hpt)
    def _zacc():
        pltpu.sync_copy(zbuf, acc_sh.at[pl.ds(s * hpt, hpt)])

    plsc.subcore_barrier()

    base = (c * NS + s) * CPT
    pltpu.sync_copy(dst_hbm.at[pl.ds(base, CPT)], dstbuf)

    ones16 = jnp.full((16,), 1.0, jnp.float32)

    @pl.loop(0, CPT)
    def _chunk(j):
        @pl.loop(0, CHUNK // 16)
        def _vec(k):
            iv = dstbuf[j, pl.ds(k * 16, 16)]
            plsc.addupdate_scatter(hist, [iv >> 7, iv & 127], ones16)

    pltpu.sync_copy(hist, acc_sh.at[idc], add=True)
    plsc.subcore_barrier()

    @pl.when(s < HROWS // hpt)
    def _wout():
        pltpu.sync_copy(acc_sh.at[pl.ds(s * hpt, hpt)],
                        out_hbm.at[c, pl.ds(s * hpt, hpt)])


_deg_kernel = functools.partial(
    pl.kernel,
    out_type=jax.ShapeDtypeStruct((NC, HROWS, CHUNK), jnp.float32),
    mesh=_sc_mesh,
    compiler_params=_cp,
    scratch_types=[
        pltpu.VMEM((CPT, CHUNK), jnp.int32),
        pltpu.VMEM((HROWS, CHUNK), jnp.float32),
        pltpu.VMEM((HROWS,), jnp.int32),
        pltpu.VMEM((8, CHUNK), jnp.float32),
        pltpu.VMEM_SHARED((HROWS, CHUNK), jnp.float32),
    ],
)(_deg_body)


def _agg_body(m2_hbm, src_hbm, dst_hbm, out_hbm,
              srcbuf, dstbuf, rows, acc_sh):
    c = lax.axis_index("c")
    s = lax.axis_index("s")

    @pl.loop(0, CHUNK)
    def _zrow(i):
        @pl.loop(0, D // 16)
        def _zcol(k):
            rows[i, pl.ds(k * 16, 16)] = _Z16()

    row0 = s * STRIP

    @pl.loop(0, STRIP // CHUNK)
    def _clear(k):
        pltpu.sync_copy(rows, acc_sh.at[pl.ds(row0 + k * CHUNK, CHUNK)])

    plsc.subcore_barrier()

    base = (c * NS + s) * CPT
    pltpu.sync_copy(src_hbm.at[pl.ds(base, CPT)], srcbuf)
    pltpu.sync_copy(dst_hbm.at[pl.ds(base, CPT)], dstbuf)

    @pl.loop(0, CPT)
    def _edge_loop(j):
        pltpu.sync_copy(m2_hbm.at[srcbuf.at[j]], rows)
        pltpu.sync_copy(rows, acc_sh.at[dstbuf.at[j]], add=True)

    plsc.subcore_barrier()

    @pl.loop(0, STRIP // CHUNK)
    def _out(k):
        pltpu.sync_copy(acc_sh.at[pl.ds(row0 + k * CHUNK, CHUNK)],
                        out_hbm.at[c, pl.ds(row0 + k * CHUNK, CHUNK)])


_agg_kernel = functools.partial(
    pl.kernel,
    out_type=jax.ShapeDtypeStruct((NC, N_PAD, D), jnp.float32),
    mesh=_sc_mesh,
    scratch_types=[
        pltpu.VMEM((CPT, CHUNK), jnp.int32),
        pltpu.VMEM((CPT, CHUNK), jnp.int32),
        pltpu.VMEM((CHUNK, D), jnp.float32),
        pltpu.VMEM_SHARED((N_PAD, D), jnp.float32),
    ],
)(_agg_body)


_DOT = functools.partial(jnp.dot, preferred_element_type=jnp.float32,
                         precision=lax.Precision.HIGHEST)


def _k1_body(x_ref, w_ref, hist_ref, m2_ref, dinv_ref):
    deg = hist_ref[0] + hist_ref[1] + 1.0
    dinv = lax.rsqrt(deg)
    dinv_ref[...] = dinv
    m = _DOT(x_ref[...], w_ref[...].T)
    m2_ref[...] = m * dinv[:N, None]


def _post_body(p_ref, m2_ref, hin_ref, b_ref, g_ref, be_ref, dinv_ref,
               w_ref, h_ref, m2n_ref=None, *, residual, matmul):
    dinv = dinv_ref[...][:N]
    t = (p_ref[0, :N, :] + p_ref[1, :N, :] + m2_ref[...]) * dinv[:, None]
    t = t + b_ref[...][None, :]
    mu = jnp.mean(t, axis=1, keepdims=True)
    var = jnp.mean((t - mu) ** 2, axis=1, keepdims=True)
    t = (t - mu) * lax.rsqrt(var + LN_EPS) * g_ref[...][None, :] \
        + be_ref[...][None, :]
    t = jnp.maximum(t, 0.0)
    if residual:
        t = t + hin_ref[...]
    h_ref[...] = t
    if matmul:
        m2n_ref[...] = _DOT(t, w_ref[...].T) * dinv[:, None]


def _tc_k1(x, w1, hist):
    return pl.pallas_call(
        _k1_body,
        out_shape=[jax.ShapeDtypeStruct((N, D), jnp.float32),
                   jax.ShapeDtypeStruct((N_PAD,), jnp.float32)],
    )(x, w1, hist)


def _tc_post(p, m2, h_in, b, g, be, dinv, w_next, residual, matmul):
    body = functools.partial(_post_body, residual=residual, matmul=matmul)
    out_shape = [jax.ShapeDtypeStruct((N, D), jnp.float32)]
    if matmul:
        out_shape.append(jax.ShapeDtypeStruct((N, D), jnp.float32))
    return pl.pallas_call(body, out_shape=out_shape)(
        p, m2, h_in, b, g, be, dinv, w_next)


def kernel(x, edge_index, Ws, bs, gammas, betas):
    src = edge_index[0].astype(jnp.int32)
    dst = edge_index[1].astype(jnp.int32)
    pad = E_PAD - E
    pad_ar = jnp.arange(pad, dtype=jnp.int32)
    pad_src = (pad_ar * 131) % N
    pad_dst = N + pad_ar % (N_PAD - N)
    srcp = jnp.concatenate([src, pad_src]).reshape(N_CHUNKS, CHUNK)
    dstp = jnp.concatenate([dst, pad_dst]).reshape(N_CHUNKS, CHUNK)

    hist = _deg_kernel(dstp).reshape(NC, N_PAD)
    m2, dinv = _tc_k1(x, Ws[0], hist)

    h = x
    for i in range(NUM_LAYERS):
        p = _agg_kernel(m2, srcp, dstp)
        last = i == NUM_LAYERS - 1
        w_next = Ws[i + 1] if not last else Ws[0]
        outs = _tc_post(p, m2, h, bs[i], gammas[i], betas[i], dinv,
                        w_next, residual=(i > 0), matmul=(not last))
        if last:
            h = outs[0]
        else:
            h, m2 = outs
    return h


# speedup vs baseline: 20.0774x; 1.0705x over previous
import dataclasses
import functools

import jax
import jax.numpy as jnp
from jax import lax
from jax.experimental import pallas as pl
from jax.experimental.pallas import tpu as pltpu
from jax.experimental.pallas import tpu_sc as plsc

N = 10000
D = 128
E = 320000
NUM_LAYERS = 3
LN_EPS = 1e-5

NC = 2
NS = 16
CHUNK = 80
N_PAD = 10240
E_PAD = 327680
N_CHUNKS = E_PAD // CHUNK
CPT = N_CHUNKS // (NC * NS)
STRIP = N_PAD // NS
HROWS = N_PAD // 128

_sc_mesh = plsc.VectorSubcoreMesh(core_axis_name="c", subcore_axis_name="s")

_cp = pltpu.CompilerParams()
if "needs_layout_passes" in pltpu.CompilerParams.__dataclass_fields__:
    _cp = dataclasses.replace(_cp, needs_layout_passes=False)

_Z16 = functools.partial(jnp.zeros, (16,), jnp.float32)


def _deg_body(dst_hbm, out_hbm, dstbuf, hist, idc, zbuf, acc_sh):
    c = lax.axis_index("c")
    s = lax.axis_index("s")
    hpt = 8

    @pl.loop(0, HROWS)
    def _zhist(i):
        @pl.loop(0, 128 // 16)
        def _zh2(j):
            hist[i, pl.ds(j * 16, 16)] = _Z16()

    @pl.loop(0, hpt)
    def _zrow(i):
        @pl.loop(0, 128 // 16)
        def _zr2(j):
            zbuf[i, pl.ds(j * 16, 16)] = _Z16()

    @pl.loop(0, HROWS // 16)
    def _iot(k):
        idc[pl.ds(k * 16, 16)] = lax.iota(jnp.int32, 16) + k * 16

    @pl.when(s < HROWS // hpt)
    def _zacc():
        pltpu.sync_copy(zbuf, acc_sh.at[pl.ds(s * hpt, hpt)])

    plsc.subcore_barrier()

    base = (c * NS + s) * CPT
    pltpu.sync_copy(dst_hbm.at[pl.ds(base, CPT)], dstbuf)

    ones16 = jnp.full((16,), 1.0, jnp.float32)

    @pl.loop(0, CPT)
    def _chunk(j):
        @pl.loop(0, CHUNK // 16)
        def _vec(k):
            iv = dstbuf[j, pl.ds(k * 16, 16)]
            plsc.addupdate_scatter(hist, [iv >> 7, iv & 127], ones16)

    pltpu.sync_copy(hist, acc_sh.at[idc], add=True)
    plsc.subcore_barrier()

    @pl.when(s < HROWS // hpt)
    def _wout():
        pltpu.sync_copy(acc_sh.at[pl.ds(s * hpt, hpt)],
                        out_hbm.at[c, pl.ds(s * hpt, hpt)])


_deg_kernel = functools.partial(
    pl.kernel,
    out_type=jax.ShapeDtypeStruct((NC, HROWS, 128), jnp.float32),
    mesh=_sc_mesh,
    compiler_params=_cp,
    scratch_types=[
        pltpu.VMEM((CPT, CHUNK), jnp.int32),
        pltpu.VMEM((HROWS, 128), jnp.float32),
        pltpu.VMEM((HROWS,), jnp.int32),
        pltpu.VMEM((8, 128), jnp.float32),
        pltpu.VMEM_SHARED((HROWS, 128), jnp.float32),
    ],
)(_deg_body)


def _agg_body(m2_hbm, src_hbm, dst_hbm, out_hbm,
              srcbuf, dstbuf, rows0, rows1, acc_sh, sem0, sem1):
    c = lax.axis_index("c")
    s = lax.axis_index("s")

    @pl.loop(0, CHUNK)
    def _zrow(i):
        @pl.loop(0, D // 16)
        def _zcol(k):
            rows0[i, pl.ds(k * 16, 16)] = _Z16()

    row0 = s * STRIP

    @pl.loop(0, STRIP // CHUNK)
    def _clear(k):
        pltpu.sync_copy(rows0, acc_sh.at[pl.ds(row0 + k * CHUNK, CHUNK)])

    plsc.subcore_barrier()

    base = (c * NS + s) * CPT
    half = CPT // 2

    for h in range(2):
        hb = base + h * half
        pltpu.sync_copy(src_hbm.at[pl.ds(hb, half)], srcbuf)
        pltpu.sync_copy(dst_hbm.at[pl.ds(hb, half)], dstbuf)
        pltpu.async_copy(m2_hbm.at[srcbuf.at[0]], rows0, sem0)

        @pl.loop(0, half // 2)
        def _edge_loop(jj):
            j0 = 2 * jj
            pltpu.make_async_copy(m2_hbm.at[srcbuf.at[j0]], rows0, sem0).wait()
            pltpu.async_copy(m2_hbm.at[srcbuf.at[j0 + 1]], rows1, sem1)
            pltpu.sync_copy(rows0, acc_sh.at[dstbuf.at[j0]], add=True)
            pltpu.make_async_copy(m2_hbm.at[srcbuf.at[j0 + 1]], rows1,
                                  sem1).wait()

            @pl.when(j0 + 2 < half)
            def _pref():
                pltpu.async_copy(m2_hbm.at[srcbuf.at[j0 + 2]], rows0, sem0)

            pltpu.sync_copy(rows1, acc_sh.at[dstbuf.at[j0 + 1]], add=True)

    plsc.subcore_barrier()

    plsc.subcore_barrier()

    @pl.loop(0, STRIP // 128)
    def _out(k):
        pltpu.sync_copy(acc_sh.at[pl.ds(row0 + k * 128, 128)],
                        out_hbm.at[c, pl.ds(row0 + k * 128, 128)])


_agg_kernel = functools.partial(
    pl.kernel,
    out_type=jax.ShapeDtypeStruct((NC, N_PAD, D), jnp.float32),
    mesh=_sc_mesh,
    compiler_params=_cp,
    scratch_types=[
        pltpu.VMEM((CPT // 2, CHUNK), jnp.int32),
        pltpu.VMEM((CPT // 2, CHUNK), jnp.int32),
        pltpu.VMEM((CHUNK, D), jnp.float32),
        pltpu.VMEM((CHUNK, D), jnp.float32),
        pltpu.VMEM_SHARED((N_PAD, D), jnp.float32),
        pltpu.SemaphoreType.DMA,
        pltpu.SemaphoreType.DMA,
    ],
)(_agg_body)


_DOT = functools.partial(jnp.dot, preferred_element_type=jnp.float32,
                         precision=lax.Precision.HIGHEST)


def _k1_body(x_ref, w_ref, hist_ref, m2_ref, dinv_ref):
    deg = hist_ref[0] + hist_ref[1] + 1.0
    dinv = lax.rsqrt(deg)
    dinv_ref[...] = dinv
    m = _DOT(x_ref[...], w_ref[...].T)
    m2_ref[...] = m * dinv[:N, None]


def _post_body(p_ref, m2_ref, hin_ref, b_ref, g_ref, be_ref, dinv_ref,
               w_ref, h_ref, m2n_ref=None, *, residual, matmul):
    dinv = dinv_ref[...][:N]
    t = (p_ref[0, :N, :] + p_ref[1, :N, :] + m2_ref[...]) * dinv[:, None]
    t = t + b_ref[...][None, :]
    mu = jnp.mean(t, axis=1, keepdims=True)
    var = jnp.mean((t - mu) ** 2, axis=1, keepdims=True)
    t = (t - mu) * lax.rsqrt(var + LN_EPS) * g_ref[...][None, :] \
        + be_ref[...][None, :]
    t = jnp.maximum(t, 0.0)
    if residual:
        t = t + hin_ref[...]
    h_ref[...] = t
    if matmul:
        m2n_ref[...] = _DOT(t, w_ref[...].T) * dinv[:, None]


def _tc_k1(x, w1, hist):
    return pl.pallas_call(
        _k1_body,
        out_shape=[jax.ShapeDtypeStruct((N, D), jnp.float32),
                   jax.ShapeDtypeStruct((N_PAD,), jnp.float32)],
    )(x, w1, hist)


def _tc_post(p, m2, h_in, b, g, be, dinv, w_next, residual, matmul):
    body = functools.partial(_post_body, residual=residual, matmul=matmul)
    out_shape = [jax.ShapeDtypeStruct((N, D), jnp.float32)]
    if matmul:
        out_shape.append(jax.ShapeDtypeStruct((N, D), jnp.float32))
    return pl.pallas_call(body, out_shape=out_shape)(
        p, m2, h_in, b, g, be, dinv, w_next)


def kernel(x, edge_index, Ws, bs, gammas, betas):
    src = edge_index[0].astype(jnp.int32)
    dst = edge_index[1].astype(jnp.int32)
    pad = E_PAD - E
    pad_ar = jnp.arange(pad, dtype=jnp.int32)
    pad_src = (pad_ar * 131) % N
    pad_dst = N + pad_ar % (N_PAD - N)
    srcp = jnp.concatenate([src, pad_src]).reshape(N_CHUNKS, CHUNK)
    dstp = jnp.concatenate([dst, pad_dst]).reshape(N_CHUNKS, CHUNK)

    hist = _deg_kernel(dstp).reshape(NC, N_PAD)
    m2, dinv = _tc_k1(x, Ws[0], hist)

    h = x
    for i in range(NUM_LAYERS):
        p = _agg_kernel(m2, srcp, dstp)
        last = i == NUM_LAYERS - 1
        w_next = Ws[i + 1] if not last else Ws[0]
        outs = _tc_post(p, m2, h, bs[i], gammas[i], betas[i], dinv,
                        w_next, residual=(i > 0), matmul=(not last))
        if last:
            h = outs[0]
        else:
            h, m2 = outs
    return h


# speedup vs baseline: 23.6961x; 1.1802x over previous
import dataclasses
import functools

import jax
import jax.numpy as jnp
from jax import lax
from jax.experimental import pallas as pl
from jax.experimental.pallas import tpu as pltpu
from jax.experimental.pallas import tpu_sc as plsc

N = 10000
D = 128
E = 320000
NUM_LAYERS = 3
LN_EPS = 1e-5

NC = 2
NS = 16
CHUNK = 128
N_PAD = 10240
E_PAD = 327680
N_CHUNKS = E_PAD // CHUNK
CPT = N_CHUNKS // (NC * NS)
STRIP = N_PAD // NS
HROWS = N_PAD // 128

_sc_mesh = plsc.VectorSubcoreMesh(core_axis_name="c", subcore_axis_name="s")

_cp = pltpu.CompilerParams()
if "needs_layout_passes" in pltpu.CompilerParams.__dataclass_fields__:
    _cp = dataclasses.replace(_cp, needs_layout_passes=False)

_Z16 = functools.partial(jnp.zeros, (16,), jnp.float32)


def _deg_body(dst_hbm, out_hbm, dstbuf, hist, idc, zbuf, acc_sh):
    c = lax.axis_index("c")
    s = lax.axis_index("s")
    hpt = 8

    @pl.loop(0, HROWS)
    def _zhist(i):
        @pl.loop(0, 128 // 16)
        def _zh2(j):
            hist[i, pl.ds(j * 16, 16)] = _Z16()

    @pl.loop(0, hpt)
    def _zrow(i):
        @pl.loop(0, 128 // 16)
        def _zr2(j):
            zbuf[i, pl.ds(j * 16, 16)] = _Z16()

    @pl.loop(0, HROWS // 16)
    def _iot(k):
        idc[pl.ds(k * 16, 16)] = lax.iota(jnp.int32, 16) + k * 16

    @pl.when(s < HROWS // hpt)
    def _zacc():
        pltpu.sync_copy(zbuf, acc_sh.at[pl.ds(s * hpt, hpt)])

    plsc.subcore_barrier()

    base = (c * NS + s) * CPT
    pltpu.sync_copy(dst_hbm.at[pl.ds(base, CPT)], dstbuf)

    ones16 = jnp.full((16,), 1.0, jnp.float32)

    @pl.loop(0, CPT)
    def _chunk(j):
        @pl.loop(0, CHUNK // 16)
        def _vec(k):
            iv = dstbuf[j, pl.ds(k * 16, 16)]
            plsc.addupdate_scatter(hist, [iv >> 7, iv & 127], ones16)

    pltpu.sync_copy(hist, acc_sh.at[idc], add=True)
    plsc.subcore_barrier()

    @pl.when(s < HROWS // hpt)
    def _wout():
        pltpu.sync_copy(acc_sh.at[pl.ds(s * hpt, hpt)],
                        out_hbm.at[c, pl.ds(s * hpt, hpt)])


_deg_kernel = functools.partial(
    pl.kernel,
    out_type=jax.ShapeDtypeStruct((NC, HROWS, 128), jnp.float32),
    mesh=_sc_mesh,
    compiler_params=_cp,
    scratch_types=[
        pltpu.VMEM((CPT, CHUNK), jnp.int32),
        pltpu.VMEM((HROWS, 128), jnp.float32),
        pltpu.VMEM((HROWS,), jnp.int32),
        pltpu.VMEM((8, 128), jnp.float32),
        pltpu.VMEM_SHARED((HROWS, 128), jnp.float32),
    ],
)(_deg_body)


def _agg_body(m2_hbm, src_hbm, dst_hbm, out_hbm,
              srcbuf, dstbuf, rows0, rows1, acc_sh, sem0, sem1):
    c = lax.axis_index("c")
    s = lax.axis_index("s")

    @pl.loop(0, CHUNK)
    def _zrow(i):
        @pl.loop(0, D // 16)
        def _zcol(k):
            rows0[i, pl.ds(k * 16, 16)] = _Z16()

    row0 = s * STRIP

    @pl.loop(0, STRIP // CHUNK)
    def _clear(k):
        pltpu.sync_copy(rows0, acc_sh.at[pl.ds(row0 + k * CHUNK, CHUNK)])

    plsc.subcore_barrier()

    base = (c * NS + s) * CPT
    half = CPT // 2

    for h in range(2):
        hb = base + h * half
        pltpu.sync_copy(src_hbm.at[pl.ds(hb, half)], srcbuf)
        pltpu.sync_copy(dst_hbm.at[pl.ds(hb, half)], dstbuf)
        pltpu.async_copy(m2_hbm.at[srcbuf.at[0]], rows0, sem0)

        @pl.loop(0, half // 2)
        def _edge_loop(jj):
            j0 = 2 * jj
            pltpu.make_async_copy(m2_hbm.at[srcbuf.at[j0]], rows0, sem0).wait()
            pltpu.async_copy(m2_hbm.at[srcbuf.at[j0 + 1]], rows1, sem1)
            pltpu.sync_copy(rows0, acc_sh.at[dstbuf.at[j0]], add=True)
            pltpu.make_async_copy(m2_hbm.at[srcbuf.at[j0 + 1]], rows1,
                                  sem1).wait()

            @pl.when(j0 + 2 < half)
            def _pref():
                pltpu.async_copy(m2_hbm.at[srcbuf.at[j0 + 2]], rows0, sem0)

            pltpu.sync_copy(rows1, acc_sh.at[dstbuf.at[j0 + 1]], add=True)

    plsc.subcore_barrier()

    plsc.subcore_barrier()

    @pl.loop(0, STRIP // 128)
    def _out(k):
        pltpu.sync_copy(acc_sh.at[pl.ds(row0 + k * 128, 128)],
                        out_hbm.at[c, pl.ds(row0 + k * 128, 128)])


_agg_kernel = functools.partial(
    pl.kernel,
    out_type=jax.ShapeDtypeStruct((NC, N_PAD, D), jnp.float32),
    mesh=_sc_mesh,
    compiler_params=_cp,
    scratch_types=[
        pltpu.VMEM((CPT // 2, CHUNK), jnp.int32),
        pltpu.VMEM((CPT // 2, CHUNK), jnp.int32),
        pltpu.VMEM((CHUNK, D), jnp.float32),
        pltpu.VMEM((CHUNK, D), jnp.float32),
        pltpu.VMEM_SHARED((N_PAD, D), jnp.float32),
        pltpu.SemaphoreType.DMA,
        pltpu.SemaphoreType.DMA,
    ],
)(_agg_body)


_DOT = functools.partial(jnp.dot, preferred_element_type=jnp.float32,
                         precision=lax.Precision.HIGHEST)


def _k1_body(x_ref, w_ref, hist_ref, m2_ref, dinv_ref):
    deg = hist_ref[0] + hist_ref[1] + 1.0
    dinv = lax.rsqrt(deg)
    dinv_ref[...] = dinv
    m = _DOT(x_ref[...], w_ref[...].T)
    m2_ref[...] = m * dinv[:N, None]


def _post_body(p_ref, m2_ref, hin_ref, b_ref, g_ref, be_ref, dinv_ref,
               w_ref, h_ref, m2n_ref=None, *, residual, matmul):
    dinv = dinv_ref[...][:N]
    t = (p_ref[0, :N, :] + p_ref[1, :N, :] + m2_ref[...]) * dinv[:, None]
    t = t + b_ref[...][None, :]
    mu = jnp.mean(t, axis=1, keepdims=True)
    var = jnp.mean((t - mu) ** 2, axis=1, keepdims=True)
    t = (t - mu) * lax.rsqrt(var + LN_EPS) * g_ref[...][None, :] \
        + be_ref[...][None, :]
    t = jnp.maximum(t, 0.0)
    if residual:
        t = t + hin_ref[...]
    h_ref[...] = t
    if matmul:
        m2n_ref[...] = _DOT(t, w_ref[...].T) * dinv[:, None]


def _tc_k1(x, w1, hist):
    return pl.pallas_call(
        _k1_body,
        out_shape=[jax.ShapeDtypeStruct((N, D), jnp.float32),
                   jax.ShapeDtypeStruct((N_PAD,), jnp.float32)],
    )(x, w1, hist)


def _tc_post(p, m2, h_in, b, g, be, dinv, w_next, residual, matmul):
    body = functools.partial(_post_body, residual=residual, matmul=matmul)
    out_shape = [jax.ShapeDtypeStruct((N, D), jnp.float32)]
    if matmul:
        out_shape.append(jax.ShapeDtypeStruct((N, D), jnp.float32))
    return pl.pallas_call(body, out_shape=out_shape)(
        p, m2, h_in, b, g, be, dinv, w_next)


def kernel(x, edge_index, Ws, bs, gammas, betas):
    src = edge_index[0].astype(jnp.int32)
    dst = edge_index[1].astype(jnp.int32)
    pad = E_PAD - E
    pad_ar = jnp.arange(pad, dtype=jnp.int32)
    pad_src = (pad_ar * 131) % N
    pad_dst = N + pad_ar % (N_PAD - N)
    srcp = jnp.concatenate([src, pad_src]).reshape(N_CHUNKS, CHUNK)
    dstp = jnp.concatenate([dst, pad_dst]).reshape(N_CHUNKS, CHUNK)

    hist = _deg_kernel(dstp).reshape(NC, N_PAD)
    m2, dinv = _tc_k1(x, Ws[0], hist)

    h = x
    for i in range(NUM_LAYERS):
        p = _agg_kernel(m2, srcp, dstp)
        last = i == NUM_LAYERS - 1
        w_next = Ws[i + 1] if not last else Ws[0]
        outs = _tc_post(p, m2, h, bs[i], gammas[i], betas[i], dinv,
                        w_next, residual=(i > 0), matmul=(not last))
        if last:
            h = outs[0]
        else:
            h, m2 = outs
    return h
